# baseline (speedup 1.0000x reference)
import jax
import jax.numpy as jnp
from jax.experimental import pallas as pl

_ROWS_PER_BLOCK = 512


def _copy_body(x_ref, o_ref):
    o_ref[...] = x_ref[...]


def kernel(inputs_embeds, embed_table):
    del embed_table
    b, s, h = inputs_embeds.shape
    x = inputs_embeds.reshape(b * s, h)
    rows = b * s
    grid = (rows // _ROWS_PER_BLOCK,)
    out = pl.pallas_call(
        _copy_body,
        grid=grid,
        in_specs=[pl.BlockSpec((_ROWS_PER_BLOCK, h), lambda i: (i, 0))],
        out_specs=pl.BlockSpec((_ROWS_PER_BLOCK, h), lambda i: (i, 0)),
        out_shape=jax.ShapeDtypeStruct((rows, h), x.dtype),
    )(x)
    return out.reshape(b, s, h)


# speedup vs baseline: 1.0108x; 1.0108x over previous
import jax
import jax.numpy as jnp
from jax.experimental import pallas as pl

_ROWS_PER_BLOCK = 1024


def _copy_body(x_ref, o_ref):
    o_ref[...] = x_ref[...]


def kernel(inputs_embeds, embed_table):
    del embed_table
    b, s, h = inputs_embeds.shape
    x = inputs_embeds.reshape(b * s, h)
    rows = b * s
    grid = (rows // _ROWS_PER_BLOCK,)
    out = pl.pallas_call(
        _copy_body,
        grid=grid,
        in_specs=[pl.BlockSpec((_ROWS_PER_BLOCK, h), lambda i: (i, 0))],
        out_specs=pl.BlockSpec((_ROWS_PER_BLOCK, h), lambda i: (i, 0)),
        out_shape=jax.ShapeDtypeStruct((rows, h), x.dtype),
    )(x)
    return out.reshape(b, s, h)
